# Initial kernel scaffold; baseline (speedup 1.0000x reference)
import jax
import jax.numpy as jnp
from jax.experimental import pallas as pl


def kernel(x, table, ln_weight, ln_bias):
    raise NotImplementedError("write your pallas kernel here")


# speedup vs baseline: 1.6347x; 1.6347x over previous
import jax
import jax.numpy as jnp
from jax import lax
from jax.experimental import pallas as pl
from jax.experimental.pallas import tpu as pltpu, tpu_sc as plsc

NC, NS, LANES = 2, 16, 16
NW = NC * NS
D = 64
CHUNK = 512
EPS = 1e-5


def _rsqrt_s(v):
    i = lax.bitcast_convert_type(v, jnp.int32)
    y = lax.bitcast_convert_type(jnp.int32(0x5F3759DF) - (i >> 1), jnp.float32)
    for _ in range(3):
        y = y * (1.5 - 0.5 * v * y * y)
    return y


def _body(x_hbm, table_hbm, w_hbm, b_hbm, out_hbm, idx_v, rows_v, w_v, b_v, sem):
    per_w = x_hbm.shape[0] // NW
    wid = lax.axis_index("s") * NC + lax.axis_index("c")
    base = wid * per_w
    pltpu.sync_copy(w_hbm, w_v)
    pltpu.sync_copy(b_hbm, b_v)
    J = D // LANES
    wregs = [w_v[pl.ds(j * LANES, LANES)] for j in range(J)]
    bregs = [b_v[pl.ds(j * LANES, LANES)] for j in range(J)]

    def chunk_body(c, _):
        start = base + c * CHUNK
        pltpu.sync_copy(x_hbm.at[pl.ds(start, CHUNK)], idx_v)
        pltpu.async_copy(table_hbm.at[idx_v], rows_v, sem).wait()

        def row(r, _):
            v = [rows_v[r, pl.ds(j * LANES, LANES)] for j in range(J)]
            s = jnp.sum((v[0] + v[1]) + (v[2] + v[3]))
            s2 = jnp.sum((v[0] * v[0] + v[1] * v[1])
                         + (v[2] * v[2] + v[3] * v[3]))
            mean = s * (1.0 / D)
            var = s2 * (1.0 / D) - mean * mean
            rstd = _rsqrt_s(var + EPS)
            mean_b = jnp.full((LANES,), mean, jnp.float32)
            rstd_b = jnp.full((LANES,), rstd, jnp.float32)
            for j in range(J):
                rows_v[r, pl.ds(j * LANES, LANES)] = (
                    (v[j] - mean_b) * (rstd_b * wregs[j]) + bregs[j])
            return 0

        lax.fori_loop(0, CHUNK, row, 0, unroll=4)
        pltpu.sync_copy(rows_v, out_hbm.at[pl.ds(start, CHUNK)])
        return 0

    lax.fori_loop(0, per_w // CHUNK, chunk_body, 0)


def kernel(x, table, ln_weight, ln_bias):
    B, L = x.shape
    n = B * L
    run = pl.kernel(
        _body,
        out_type=jax.ShapeDtypeStruct((n, D), jnp.float32),
        mesh=plsc.VectorSubcoreMesh(
            core_axis_name="c", subcore_axis_name="s",
            num_cores=NC, num_subcores=NS,
        ),
        scratch_types=[
            pltpu.VMEM((CHUNK,), jnp.int32),
            pltpu.VMEM((CHUNK, D), jnp.float32),
            pltpu.VMEM((D,), jnp.float32),
            pltpu.VMEM((D,), jnp.float32),
            pltpu.SemaphoreType.DMA,
        ],
        compiler_params=pltpu.CompilerParams(
            needs_layout_passes=False, use_tc_tiling_on_sc=False),
    )
    out = run(x.reshape(-1), table, ln_weight, ln_bias)
    return out.reshape(B, L, D)
